# baseline (device time: 231462 ns/iter reference)
import jax
import jax.numpy as jnp
from jax import lax
from jax.experimental import pallas as pl
from jax.experimental.pallas import tpu as pltpu

N_DEV = 4


def kernel(table, idx):
    v_per, d = table.shape
    n = idx.shape[0]

    my = lax.axis_index("i")
    local = idx - my * v_per
    mask = (local >= 0) & (local < v_per)
    safe = jnp.where(mask, local, 0)
    part = jnp.where(mask[:, None], table[safe, :], 0.0).astype(jnp.bfloat16)

    def body(p_ref, out_ref, comm_ref, send_sems, recv_sems):
        my_pos = lax.axis_index("i")
        left = (my_pos - 1) % N_DEV
        right = (my_pos + 1) % N_DEV

        barrier_sem = pltpu.get_barrier_semaphore()
        for nbr in (left, right):
            pl.semaphore_signal(
                barrier_sem, inc=1,
                device_id=(nbr,), device_id_type=pl.DeviceIdType.MESH,
            )
        pl.semaphore_wait(barrier_sem, 2)

        comm_ref[0] = p_ref[...]
        out_ref[...] = p_ref[...]

        for h in range(N_DEV - 1):
            rdma = pltpu.make_async_remote_copy(
                src_ref=comm_ref.at[h],
                dst_ref=comm_ref.at[h + 1],
                send_sem=send_sems.at[h],
                recv_sem=recv_sems.at[h],
                device_id=(right,),
                device_id_type=pl.DeviceIdType.MESH,
            )
            rdma.start()
            rdma.wait()
            out_ref[...] += comm_ref[h + 1]

    return pl.pallas_call(
        body,
        out_shape=jax.ShapeDtypeStruct((n, d), jnp.bfloat16),
        in_specs=[pl.BlockSpec(memory_space=pltpu.VMEM)],
        out_specs=pl.BlockSpec(memory_space=pltpu.VMEM),
        scratch_shapes=[
            pltpu.VMEM((N_DEV, n, d), jnp.bfloat16),
            pltpu.SemaphoreType.DMA((N_DEV - 1,)),
            pltpu.SemaphoreType.DMA((N_DEV - 1,)),
        ],
        compiler_params=pltpu.CompilerParams(collective_id=0),
    )(part)


# device time: 154675 ns/iter; 1.4964x vs baseline; 1.4964x over previous
import jax
import jax.numpy as jnp
from jax import lax
from jax.experimental import pallas as pl
from jax.experimental.pallas import tpu as pltpu

N_DEV = 4
BLK = 128


def kernel(table, idx):
    v_per, d = table.shape
    n = idx.shape[0]
    chunk = n // N_DEV

    def body(idx_ref, table_ref, out_ref, part_ref, comm_ref,
             dma_sem, send_sems, recv_sems):
        my_pos = lax.axis_index("i")
        left = (my_pos - 1) % N_DEV
        right = (my_pos + 1) % N_DEV
        base = my_pos * v_per

        barrier_sem = pltpu.get_barrier_semaphore()
        for nbr in (left, right):
            pl.semaphore_signal(
                barrier_sem, inc=1,
                device_id=(nbr,), device_id_type=pl.DeviceIdType.MESH,
            )

        part_ref[...] = jnp.zeros((n, d), jnp.float32)

        def row_copy(i):
            l = idx_ref[i] - base
            owned = jnp.logical_and(l >= 0, l < v_per)
            li = jnp.clip(l, 0, v_per - 1)
            return owned, pltpu.make_async_copy(
                table_ref.at[li], part_ref.at[i], dma_sem
            )

        def issue_blk(b):
            def it(i, _):
                owned, cp = row_copy(i)
                @pl.when(owned)
                def _():
                    cp.start()
                return _
            lax.fori_loop(b * BLK, (b + 1) * BLK, it, None)

        def wait_blk(b):
            def it(i, _):
                owned, cp = row_copy(i)
                @pl.when(owned)
                def _():
                    cp.wait()
                return _
            lax.fori_loop(b * BLK, (b + 1) * BLK, it, None)

        nblk = n // BLK
        issue_blk(0)
        for b in range(1, nblk):
            issue_blk(b)
            wait_blk(b - 1)
        wait_blk(nblk - 1)

        out_ref[...] = part_ref[...].astype(jnp.bfloat16)

        pl.semaphore_wait(barrier_sem, 2)

        for s in range(N_DEV - 1):
            send_c = (my_pos - s) % N_DEV
            recv_c = (my_pos - s - 1) % N_DEV
            rdma = pltpu.make_async_remote_copy(
                src_ref=out_ref.at[pl.ds(send_c * chunk, chunk), :],
                dst_ref=comm_ref.at[s],
                send_sem=send_sems.at[s],
                recv_sem=recv_sems.at[s],
                device_id=(right,),
                device_id_type=pl.DeviceIdType.MESH,
            )
            rdma.start()
            rdma.wait()
            out_ref[pl.ds(recv_c * chunk, chunk), :] += comm_ref[s]

        for s in range(N_DEV - 1):
            send_c = (my_pos + 1 - s) % N_DEV
            rdma = pltpu.make_async_remote_copy(
                src_ref=out_ref.at[pl.ds(send_c * chunk, chunk), :],
                dst_ref=out_ref.at[pl.ds(send_c * chunk, chunk), :],
                send_sem=send_sems.at[N_DEV - 1 + s],
                recv_sem=recv_sems.at[N_DEV - 1 + s],
                device_id=(right,),
                device_id_type=pl.DeviceIdType.MESH,
            )
            rdma.start()
            rdma.wait()

    return pl.pallas_call(
        body,
        out_shape=jax.ShapeDtypeStruct((n, d), jnp.bfloat16),
        in_specs=[
            pl.BlockSpec(memory_space=pltpu.SMEM),
            pl.BlockSpec(memory_space=pl.ANY),
        ],
        out_specs=pl.BlockSpec(memory_space=pltpu.VMEM),
        scratch_shapes=[
            pltpu.VMEM((n, d), jnp.float32),
            pltpu.VMEM((N_DEV - 1, chunk, d), jnp.bfloat16),
            pltpu.SemaphoreType.DMA,
            pltpu.SemaphoreType.DMA((2 * (N_DEV - 1),)),
            pltpu.SemaphoreType.DMA((2 * (N_DEV - 1),)),
        ],
        compiler_params=pltpu.CompilerParams(collective_id=0),
    )(idx, table)


# device time: 67560 ns/iter; 3.4260x vs baseline; 2.2894x over previous
import jax
import jax.numpy as jnp
from jax import lax
from jax.experimental import pallas as pl
from jax.experimental.pallas import tpu as pltpu

N_DEV = 4


def kernel(table, idx):
    v_per, d = table.shape
    n = idx.shape[0]
    chunk = n // N_DEV
    cbits = chunk.bit_length()

    my = lax.axis_index("i")
    local = idx - my * v_per
    owned = (local >= 0) & (local < v_per)
    li = jnp.clip(local, 0, v_per - 1).astype(jnp.int32)
    ow = owned.astype(jnp.int32)
    cnts = ow.reshape(N_DEV, chunk).sum(axis=1, dtype=jnp.int32)
    maskf = owned.astype(jnp.float32).reshape(n, 1)

    def body(li_ref, ow_ref, cnt_ref, m_ref, table_ref, out_ref,
             part_ref, comm_ref, dma_sems, send_sems, recv_sems):
        my_pos = lax.axis_index("i")
        left = (my_pos - 1) % N_DEV
        right = (my_pos + 1) % N_DEV

        barrier_sem = pltpu.get_barrier_semaphore()
        for nbr in (left, right):
            pl.semaphore_signal(
                barrier_sem, inc=1,
                device_id=(nbr,), device_id_type=pl.DeviceIdType.MESH,
            )

        def prep(c, t):
            lo = c * chunk

            def issue(j, _):
                i = lo + j
                @pl.when(ow_ref[i] == 1)
                def _():
                    pltpu.make_async_copy(
                        table_ref.at[li_ref[i]], part_ref.at[i], dma_sems.at[t]
                    ).start()
                return _
            lax.fori_loop(0, chunk, issue, None, unroll=16)

        def finish(c, t):
            lo = c * chunk
            cnt = cnt_ref[c]
            for k in range(cbits):
                @pl.when(((cnt >> k) & 1) == 1)
                def _():
                    pltpu.make_async_copy(
                        table_ref.at[pl.ds(0, 1 << k), :],
                        part_ref.at[pl.ds(lo, 1 << k), :],
                        dma_sems.at[t],
                    ).wait()
            sl = pl.ds(lo, chunk)
            out_ref[sl, :] = jnp.where(
                m_ref[sl, :] > 0, part_ref[sl, :], 0.0
            ).astype(jnp.bfloat16)

        d2 = d // 2
        A = pl.ds(0, d2)
        B = pl.ds(d2, d2)

        def rs_rdma(send_c, half, buf_i, sem_i, dev):
            return pltpu.make_async_remote_copy(
                src_ref=out_ref.at[pl.ds(send_c * chunk, chunk), half],
                dst_ref=comm_ref.at[buf_i],
                send_sem=send_sems.at[sem_i],
                recv_sem=recv_sems.at[sem_i],
                device_id=(dev,),
                device_id_type=pl.DeviceIdType.MESH,
            )

        def ag_rdma(send_c, half, sem_i, dev):
            return pltpu.make_async_remote_copy(
                src_ref=out_ref.at[pl.ds(send_c * chunk, chunk), half],
                dst_ref=out_ref.at[pl.ds(send_c * chunk, chunk), half],
                send_sem=send_sems.at[sem_i],
                recv_sem=recv_sems.at[sem_i],
                device_id=(dev,),
                device_id_type=pl.DeviceIdType.MESH,
            )

        c_p1 = (my_pos + 1) % N_DEV
        c_p2 = (my_pos + 2) % N_DEV
        c_p3 = (my_pos + 3) % N_DEV

        prep(c_p2, 0)
        finish(c_p2, 0)
        pl.semaphore_wait(barrier_sem, 2)
        r0 = rs_rdma(c_p2, A, 0, 0, right)
        l0 = rs_rdma(c_p2, B, 1, 1, left)
        r0.start()
        l0.start()
        prep(c_p1, 1)
        finish(c_p1, 1)
        dr = rs_rdma(c_p1, B, 2, 2, right)
        dr.start()
        prep(c_p3, 2)
        finish(c_p3, 2)
        dl = rs_rdma(c_p3, A, 3, 3, left)
        dl.start()
        r0.wait()
        out_ref[pl.ds(c_p1 * chunk, chunk), A] += comm_ref[0]
        r1 = rs_rdma(c_p1, A, 4, 4, right)
        r1.start()
        l0.wait()
        out_ref[pl.ds(c_p3 * chunk, chunk), B] += comm_ref[1]
        l1 = rs_rdma(c_p3, B, 5, 5, left)
        l1.start()
        prep(my_pos, 3)
        finish(my_pos, 3)
        a0 = ag_rdma(my_pos, A, 6, right)
        b0 = ag_rdma(my_pos, B, 7, left)
        al = ag_rdma(my_pos, A, 8, left)
        br = ag_rdma(my_pos, B, 9, right)
        r1.wait()
        dl.wait()
        out_ref[pl.ds(my_pos * chunk, chunk), A] += (
            comm_ref[4] + comm_ref[3]
        )
        a0.start()
        al.start()
        l1.wait()
        dr.wait()
        out_ref[pl.ds(my_pos * chunk, chunk), B] += (
            comm_ref[5] + comm_ref[2]
        )
        b0.start()
        br.start()
        a0.wait()
        a1 = ag_rdma(c_p3, A, 10, right)
        a1.start()
        b0.wait()
        b1 = ag_rdma(c_p1, B, 11, left)
        b1.start()
        al.wait()
        br.wait()
        a1.wait()
        b1.wait()

    return pl.pallas_call(
        body,
        out_shape=jax.ShapeDtypeStruct((n, d), jnp.bfloat16),
        in_specs=[
            pl.BlockSpec(memory_space=pltpu.SMEM),
            pl.BlockSpec(memory_space=pltpu.SMEM),
            pl.BlockSpec(memory_space=pltpu.SMEM),
            pl.BlockSpec(memory_space=pltpu.VMEM),
            pl.BlockSpec(memory_space=pl.ANY),
        ],
        out_specs=pl.BlockSpec(memory_space=pltpu.VMEM),
        scratch_shapes=[
            pltpu.VMEM((n, d), jnp.float32),
            pltpu.VMEM((6, chunk, d // 2), jnp.bfloat16),
            pltpu.SemaphoreType.DMA((N_DEV,)),
            pltpu.SemaphoreType.DMA((12,)),
            pltpu.SemaphoreType.DMA((12,)),
        ],
        compiler_params=pltpu.CompilerParams(collective_id=0),
    )(li, ow, cnts, maskf, table)
